# Initial kernel scaffold; baseline (speedup 1.0000x reference)
import jax
import jax.numpy as jnp
from jax.experimental import pallas as pl


def kernel(x, enc_W, enc_b, l0_Wl, l0_bl, l0_Wr, h1_Wl, h1_bl, h1_Wr, h2_Wl, h2_bl, h2_Wr, out_Wl, out_bl, out_Wr, dec_W, dec_b, edge_index):
    raise NotImplementedError("write your pallas kernel here")


# speedup vs baseline: 5.1905x; 5.1905x over previous
import jax
import jax.numpy as jnp
from jax import lax
from jax.experimental import pallas as pl
from jax.experimental.pallas import tpu as pltpu
from jax.experimental.pallas import tpu_sc as plsc

N = 10000
E = 320000
H = 128

NC = 2
NS = 16
NW = NC * NS
EPW = E // NW
B = 80
NCHUNK = EPW // B
NP = 10240
RPS = NP // NS

ROWS_BLK = 1000
GRID_N = N // ROWS_BLK


_MESH = plsc.VectorSubcoreMesh(core_axis_name="c", subcore_axis_name="s",
                               num_cores=NC, num_subcores=NS)
_UNTILED = pltpu.CompilerParams(use_tc_tiling_on_sc=False)


def _make_sc_agg(width):
    out_type = jax.ShapeDtypeStruct((NC, NP, width), jnp.float32)
    scratch = [
        pltpu.VMEM((B,), jnp.int32),
        pltpu.VMEM((B,), jnp.int32),
        pltpu.VMEM((B, width), jnp.float32),
        pltpu.VMEM_SHARED((NP, width), jnp.float32),
        pltpu.SemaphoreType.DMA,
    ]

    def body(table, src, dst, zrows, parts, srcv, dstv, rows, acc, sem):
        c = lax.axis_index("c")
        s = lax.axis_index("s")
        base = (s * NC + c) * EPW
        myrows = pl.ds(s * RPS, RPS)
        pltpu.sync_copy(zrows, acc.at[myrows])
        plsc.subcore_barrier()

        def chunk(i, carry):
            off = base + i * B
            pltpu.sync_copy(src.at[pl.ds(off, B)], srcv)
            pltpu.sync_copy(dst.at[pl.ds(off, B)], dstv)
            pltpu.async_copy(table.at[srcv], rows, sem).wait()
            pltpu.sync_copy(rows, acc.at[dstv], add=True)
            return carry

        lax.fori_loop(0, NCHUNK, chunk, 0)
        plsc.subcore_barrier()
        pltpu.sync_copy(acc.at[myrows], parts.at[c, myrows])

    params = _UNTILED if width % 128 != 0 else None
    return pl.kernel(body, out_type=out_type, mesh=_MESH,
                     scratch_types=tuple(scratch), compiler_params=params)


def _make_sc_deg():
    out_type = jax.ShapeDtypeStruct((NC, NP, 16), jnp.float32)
    scratch = [
        pltpu.VMEM((B,), jnp.int32),
        pltpu.VMEM((B, 16), jnp.float32),
        pltpu.VMEM_SHARED((NP, 16), jnp.float32),
    ]

    def body(dst, zrows16, ones16, degparts, dstv, onesv, accd):
        c = lax.axis_index("c")
        s = lax.axis_index("s")
        base = (s * NC + c) * EPW
        myrows = pl.ds(s * RPS, RPS)
        pltpu.sync_copy(zrows16, accd.at[myrows])
        pltpu.sync_copy(ones16, onesv)
        plsc.subcore_barrier()

        def chunk(i, carry):
            off = base + i * B
            pltpu.sync_copy(dst.at[pl.ds(off, B)], dstv)
            pltpu.sync_copy(onesv, accd.at[dstv], add=True)
            return carry

        lax.fori_loop(0, NCHUNK, chunk, 0)
        plsc.subcore_barrier()
        pltpu.sync_copy(accd.at[myrows], degparts.at[c, myrows])

    return pl.kernel(body, out_type=out_type, mesh=_MESH,
                     scratch_types=tuple(scratch), compiler_params=_UNTILED)


def _dot_t(a, w):
    return lax.dot_general(a, w, (((1,), (1,)), ((), ())),
                           preferred_element_type=jnp.float32)


def _encode_body(x_ref, w_ref, b_ref, o_ref):
    o_ref[...] = jnp.maximum(_dot_t(x_ref[...], w_ref[...]) + b_ref[...], 0.0)


def _combine_body(p_ref, dp_ref, z_ref, wl_ref, bl_ref, wr_ref, o_ref):
    deg = dp_ref[0, :, 0:1] + dp_ref[1, :, 0:1]
    inv = 1.0 / jnp.maximum(deg, 1.0)
    mean = (p_ref[0] + p_ref[1]) * inv
    acc = _dot_t(mean, wl_ref[...]) + bl_ref[...] + _dot_t(z_ref[...], wr_ref[...])
    o_ref[...] = jnp.maximum(acc, 0.0)


def _prep3_body(z_ref, wl_ref, wr_ref, p_ref, r_ref):
    p_ref[...] = _dot_t(z_ref[...], wl_ref[...])
    r_ref[...] = _dot_t(z_ref[...], wr_ref[...])


def _combine3_body(p_ref, dp_ref, r_ref, bl_ref, dw_ref, db_ref, o_ref):
    deg = dp_ref[0, :, 0:1] + dp_ref[1, :, 0:1]
    inv = 1.0 / jnp.maximum(deg, 1.0)
    z4 = (p_ref[0] + p_ref[1]) * inv + bl_ref[...] + r_ref[...]
    o_ref[...] = _dot_t(z4, dw_ref[...]) + db_ref[...]


def _row_spec(width):
    return pl.BlockSpec((ROWS_BLK, width), lambda i: (i, 0))


def _part_spec(width):
    return pl.BlockSpec((NC, ROWS_BLK, width), lambda i: (0, i, 0))


def _full_spec(shape):
    return pl.BlockSpec(shape, lambda i: tuple(0 for _ in shape))


def _tc_call(body, in_specs, out_specs, out_shape):
    return pl.pallas_call(body, grid=(GRID_N,), in_specs=in_specs,
                          out_specs=out_specs, out_shape=out_shape)


@jax.jit
def kernel(x, enc_W, enc_b, l0_Wl, l0_bl, l0_Wr, h1_Wl, h1_bl, h1_Wr,
           h2_Wl, h2_bl, h2_Wr, out_Wl, out_bl, out_Wr, dec_W, dec_b,
           edge_index):
    src = edge_index[0]
    dst = edge_index[1]
    zrows = jnp.zeros((RPS, H), jnp.float32)
    zrows16 = jnp.zeros((RPS, 16), jnp.float32)
    ones16 = jnp.ones((B, 16), jnp.float32)

    agg128 = _make_sc_agg(H)
    agg16 = _make_sc_agg(16)
    deg16 = _make_sc_deg()

    f32 = jnp.float32
    enc = _tc_call(
        _encode_body,
        [_row_spec(H), _full_spec((H, H)), _full_spec((1, H))],
        _row_spec(H), jax.ShapeDtypeStruct((N, H), f32))
    combine = _tc_call(
        _combine_body,
        [_part_spec(H), _part_spec(16), _row_spec(H),
         _full_spec((H, H)), _full_spec((1, H)), _full_spec((H, H))],
        _row_spec(H), jax.ShapeDtypeStruct((N, H), f32))
    prep3 = _tc_call(
        _prep3_body,
        [_row_spec(H), _full_spec((16, H)), _full_spec((16, H))],
        [_row_spec(16), _row_spec(16)],
        [jax.ShapeDtypeStruct((N, 16), f32), jax.ShapeDtypeStruct((N, 16), f32)])
    combine3 = _tc_call(
        _combine3_body,
        [_part_spec(16), _part_spec(16), _row_spec(16),
         _full_spec((1, 16)), _full_spec((16, 16)), _full_spec((1, 16))],
        _row_spec(16), jax.ShapeDtypeStruct((N, 16), f32))

    z = enc(x, enc_W, enc_b.reshape(1, H))
    degparts = deg16(dst, zrows16, ones16)
    parts = agg128(z, src, dst, zrows)
    z = combine(parts, degparts, z, l0_Wl, l0_bl.reshape(1, H), l0_Wr)
    parts = agg128(z, src, dst, zrows)
    z = combine(parts, degparts, z, h1_Wl, h1_bl.reshape(1, H), h1_Wr)
    parts = agg128(z, src, dst, zrows)
    z = combine(parts, degparts, z, h2_Wl, h2_bl.reshape(1, H), h2_Wr)
    wl4 = jnp.zeros((16, H), f32).at[:4].set(out_Wl)
    wr4 = jnp.zeros((16, H), f32).at[:4].set(out_Wr)
    p3, r3 = prep3(z, wl4, wr4)
    parts = agg16(p3, src, dst, zrows16)
    bl4 = jnp.zeros((1, 16), f32).at[0, :4].set(out_bl)
    dw = jnp.zeros((16, 16), f32).at[:4, :4].set(dec_W)
    db = jnp.zeros((1, 16), f32).at[0, :4].set(dec_b)
    out16 = combine3(parts, degparts, r3, bl4, dw, db)
    return out16[:, :4]


# speedup vs baseline: 9.5110x; 1.8324x over previous
import jax
import jax.numpy as jnp
from jax import lax
from jax.experimental import pallas as pl
from jax.experimental.pallas import tpu as pltpu
from jax.experimental.pallas import tpu_sc as plsc

N = 10000
E = 320000
H = 128

NC = 2
NS = 16
NW = NC * NS
EPW = E // NW
B = 40
NCHUNK = EPW // B
NP = 10240
RPS = NP // NS

ROWS_BLK = 1000
GRID_N = N // ROWS_BLK


_MESH = plsc.VectorSubcoreMesh(core_axis_name="c", subcore_axis_name="s",
                               num_cores=NC, num_subcores=NS)
_UNTILED = pltpu.CompilerParams(use_tc_tiling_on_sc=False)


NB = 5


def _make_sc_agg(width):
    out_type = jax.ShapeDtypeStruct((NC, NP, width), jnp.float32)
    scratch = (
        [pltpu.VMEM((NB, 2, B), jnp.int32),
         pltpu.VMEM((NB, B, width), jnp.float32),
         pltpu.VMEM_SHARED((NP, width), jnp.float32)]
        + [pltpu.SemaphoreType.DMA] * (4 * NB)
    )

    def body(table, esrc, edst, zrows, parts, ev, rows, acc, *sems):
        semi = sems[:2 * NB]
        semg, sema = sems[2 * NB:3 * NB], sems[3 * NB:]
        c = lax.axis_index("c")
        s = lax.axis_index("s")
        base = (s * NC + c) * EPW
        myrows = pl.ds(s * RPS, RPS)
        pltpu.sync_copy(zrows, acc.at[myrows])
        plsc.subcore_barrier()

        def group(g, carry):
            first = base + g * (NB * B)
            di = []
            for b in range(NB):
                sl = pl.ds(first + b * B, B)
                di.append((pltpu.async_copy(esrc.at[sl], ev.at[b, 0],
                                            semi[2 * b]),
                           pltpu.async_copy(edst.at[sl], ev.at[b, 1],
                                            semi[2 * b + 1])))
            dg = []
            for b in range(NB):
                di[b][0].wait()
                dg.append(pltpu.async_copy(table.at[ev.at[b, 0]],
                                           rows.at[b], semg[b]))
            da = []
            for b in range(NB):
                dg[b].wait()
                di[b][1].wait()
                da.append(pltpu.async_copy(rows.at[b], acc.at[ev.at[b, 1]],
                                           sema[b], add=True))
            for b in range(NB):
                da[b].wait()
            return carry

        lax.fori_loop(0, NCHUNK // NB, group, 0)
        plsc.subcore_barrier()
        pltpu.sync_copy(acc.at[myrows], parts.at[c, myrows])

    params = _UNTILED if width % 128 != 0 else None
    return pl.kernel(body, out_type=out_type, mesh=_MESH,
                     scratch_types=tuple(scratch), compiler_params=params)


def _make_sc_deg():
    out_type = jax.ShapeDtypeStruct((NC, NP, 16), jnp.float32)
    scratch = (
        [pltpu.VMEM((NB, 1, B), jnp.int32),
         pltpu.VMEM((B, 16), jnp.float32),
         pltpu.VMEM_SHARED((NP, 16), jnp.float32)]
        + [pltpu.SemaphoreType.DMA] * (2 * NB)
    )

    def body(dst, zrows16, ones16, degparts, dstv, onesv, accd, *sems):
        semi, sema = sems[:NB], sems[NB:]
        c = lax.axis_index("c")
        s = lax.axis_index("s")
        base = (s * NC + c) * EPW
        myrows = pl.ds(s * RPS, RPS)
        pltpu.sync_copy(zrows16, accd.at[myrows])
        pltpu.sync_copy(ones16, onesv)
        plsc.subcore_barrier()

        def group(g, carry):
            first = base + g * (NB * B)
            di = [pltpu.async_copy(dst.at[pl.ds(first + b * B, B)],
                                   dstv.at[b, 0], semi[b]) for b in range(NB)]
            da = []
            for b in range(NB):
                di[b].wait()
                da.append(pltpu.async_copy(onesv, accd.at[dstv.at[b, 0]],
                                           sema[b], add=True))
            for b in range(NB):
                da[b].wait()
            return carry

        lax.fori_loop(0, NCHUNK // NB, group, 0)
        plsc.subcore_barrier()
        pltpu.sync_copy(accd.at[myrows], degparts.at[c, myrows])

    return pl.kernel(body, out_type=out_type, mesh=_MESH,
                     scratch_types=tuple(scratch), compiler_params=_UNTILED)


def _dot_t(a, w):
    return lax.dot_general(a, w, (((1,), (1,)), ((), ())),
                           preferred_element_type=jnp.float32)


def _encode_body(x_ref, w_ref, b_ref, o_ref):
    o_ref[...] = jnp.maximum(_dot_t(x_ref[...], w_ref[...]) + b_ref[...], 0.0)


def _combine_body(p_ref, dp_ref, z_ref, wl_ref, bl_ref, wr_ref, o_ref):
    deg = dp_ref[0, :, 0:1] + dp_ref[1, :, 0:1]
    inv = 1.0 / jnp.maximum(deg, 1.0)
    mean = (p_ref[0] + p_ref[1]) * inv
    acc = _dot_t(mean, wl_ref[...]) + bl_ref[...] + _dot_t(z_ref[...], wr_ref[...])
    o_ref[...] = jnp.maximum(acc, 0.0)


def _prep3_body(z_ref, wl_ref, wr_ref, p_ref, r_ref):
    p_ref[...] = _dot_t(z_ref[...], wl_ref[...])
    r_ref[...] = _dot_t(z_ref[...], wr_ref[...])


def _combine3_body(p_ref, dp_ref, r_ref, bl_ref, dw_ref, db_ref, o_ref):
    deg = dp_ref[0, :, 0:1] + dp_ref[1, :, 0:1]
    inv = 1.0 / jnp.maximum(deg, 1.0)
    z4 = (p_ref[0] + p_ref[1]) * inv + bl_ref[...] + r_ref[...]
    o_ref[...] = _dot_t(z4, dw_ref[...]) + db_ref[...]


def _row_spec(width):
    return pl.BlockSpec((ROWS_BLK, width), lambda i: (i, 0))


def _part_spec(width):
    return pl.BlockSpec((NC, ROWS_BLK, width), lambda i: (0, i, 0))


def _full_spec(shape):
    return pl.BlockSpec(shape, lambda i: tuple(0 for _ in shape))


def _tc_call(body, in_specs, out_specs, out_shape):
    return pl.pallas_call(body, grid=(GRID_N,), in_specs=in_specs,
                          out_specs=out_specs, out_shape=out_shape)


@jax.jit
def kernel(x, enc_W, enc_b, l0_Wl, l0_bl, l0_Wr, h1_Wl, h1_bl, h1_Wr,
           h2_Wl, h2_bl, h2_Wr, out_Wl, out_bl, out_Wr, dec_W, dec_b,
           edge_index):
    src = edge_index[0]
    dst = edge_index[1]
    zrows = jnp.zeros((RPS, H), jnp.float32)
    zrows16 = jnp.zeros((RPS, 16), jnp.float32)
    ones16 = jnp.ones((B, 16), jnp.float32)

    agg128 = _make_sc_agg(H)
    agg16 = _make_sc_agg(16)
    deg16 = _make_sc_deg()

    f32 = jnp.float32
    enc = _tc_call(
        _encode_body,
        [_row_spec(H), _full_spec((H, H)), _full_spec((1, H))],
        _row_spec(H), jax.ShapeDtypeStruct((N, H), f32))
    combine = _tc_call(
        _combine_body,
        [_part_spec(H), _part_spec(16), _row_spec(H),
         _full_spec((H, H)), _full_spec((1, H)), _full_spec((H, H))],
        _row_spec(H), jax.ShapeDtypeStruct((N, H), f32))
    prep3 = _tc_call(
        _prep3_body,
        [_row_spec(H), _full_spec((16, H)), _full_spec((16, H))],
        [_row_spec(16), _row_spec(16)],
        [jax.ShapeDtypeStruct((N, 16), f32), jax.ShapeDtypeStruct((N, 16), f32)])
    combine3 = _tc_call(
        _combine3_body,
        [_part_spec(16), _part_spec(16), _row_spec(16),
         _full_spec((1, 16)), _full_spec((16, 16)), _full_spec((1, 16))],
        _row_spec(16), jax.ShapeDtypeStruct((N, 16), f32))

    z = enc(x, enc_W, enc_b.reshape(1, H))
    degparts = deg16(dst, zrows16, ones16)
    parts = agg128(z, src, dst, zrows)
    z = combine(parts, degparts, z, l0_Wl, l0_bl.reshape(1, H), l0_Wr)
    parts = agg128(z, src, dst, zrows)
    z = combine(parts, degparts, z, h1_Wl, h1_bl.reshape(1, H), h1_Wr)
    parts = agg128(z, src, dst, zrows)
    z = combine(parts, degparts, z, h2_Wl, h2_bl.reshape(1, H), h2_Wr)
    wl4 = jnp.zeros((16, H), f32).at[:4].set(out_Wl)
    wr4 = jnp.zeros((16, H), f32).at[:4].set(out_Wr)
    p3, r3 = prep3(z, wl4, wr4)
    parts = agg16(p3, src, dst, zrows16)
    bl4 = jnp.zeros((1, 16), f32).at[0, :4].set(out_bl)
    dw = jnp.zeros((16, 16), f32).at[:4, :4].set(dec_W)
    db = jnp.zeros((1, 16), f32).at[0, :4].set(dec_b)
    out16 = combine3(parts, degparts, r3, bl4, dw, db)
    return out16[:, :4]


# speedup vs baseline: 13.6797x; 1.4383x over previous
import jax
import jax.numpy as jnp
from jax import lax
from jax.experimental import pallas as pl
from jax.experimental.pallas import tpu as pltpu
from jax.experimental.pallas import tpu_sc as plsc

N = 10000
E = 320000
H = 128

NC = 2
NS = 16
NW = NC * NS
EPW = E // NW
B = 40
B16 = 80
NP = 10240
RPS = NP // NS

ROWS_BLK = 1000
GRID_N = N // ROWS_BLK


_MESH = plsc.VectorSubcoreMesh(core_axis_name="c", subcore_axis_name="s",
                               num_cores=NC, num_subcores=NS)
_UNTILED = pltpu.CompilerParams(use_tc_tiling_on_sc=False)


NB = 5


def _make_sc_agg(width, bb, nb=NB):
    nchunk = EPW // bb
    out_type = jax.ShapeDtypeStruct((NC, NP, width), jnp.float32)
    scratch = (
        [pltpu.VMEM((nchunk, bb), jnp.int32),
         pltpu.VMEM((nchunk, bb), jnp.int32),
         pltpu.VMEM((nb, bb, width), jnp.float32),
         pltpu.VMEM_SHARED((NP, width), jnp.float32)]
        + [pltpu.SemaphoreType.DMA] * (2 * nb)
    )

    def body(table, src_r, dst_r, zrows, parts, evs, evd, rows, acc, *sems):
        semg, sema = sems[:nb], sems[nb:]
        c = lax.axis_index("c")
        s = lax.axis_index("s")
        w = s * NC + c
        myrows = pl.ds(s * RPS, RPS)
        pltpu.sync_copy(src_r.at[w], evs)
        pltpu.sync_copy(dst_r.at[w], evd)
        pltpu.sync_copy(zrows, acc.at[myrows])
        plsc.subcore_barrier()

        def group(g, carry):
            i0 = g * nb

            dg = []
            for b in range(nb):
                @pl.when(g > 0)
                def _(b=b):
                    pltpu.make_async_copy(
                        rows.at[b], acc.at[evd.at[i0 - nb + b]],
                        sema[b]).wait()
                dg.append(pltpu.async_copy(table.at[evs.at[i0 + b]],
                                           rows.at[b], semg[b]))
            for b in range(nb):
                dg[b].wait()
                pltpu.async_copy(rows.at[b], acc.at[evd.at[i0 + b]],
                                 sema[b], add=True)
            return carry

        lax.fori_loop(0, nchunk // nb, group, 0)
        for b in range(nb):
            pltpu.make_async_copy(rows.at[b], acc.at[evd.at[nchunk - nb + b]],
                                  sema[b]).wait()
        plsc.subcore_barrier()
        pltpu.sync_copy(acc.at[myrows], parts.at[c, myrows])

    return pl.kernel(body, out_type=out_type, mesh=_MESH,
                     scratch_types=tuple(scratch), compiler_params=_UNTILED)


def _make_sc_deg():
    nchunk = EPW // B16
    out_type = jax.ShapeDtypeStruct((NC, NP, 16), jnp.float32)
    scratch = (
        [pltpu.VMEM((nchunk, B16), jnp.int32),
         pltpu.VMEM((B16, 16), jnp.float32),
         pltpu.VMEM_SHARED((NP, 16), jnp.float32)]
        + [pltpu.SemaphoreType.DMA] * NB
    )

    def body(dst_r, zrows16, ones16, degparts, evd, onesv, accd, *sems):
        c = lax.axis_index("c")
        s = lax.axis_index("s")
        w = s * NC + c
        myrows = pl.ds(s * RPS, RPS)
        pltpu.sync_copy(dst_r.at[w], evd)
        pltpu.sync_copy(zrows16, accd.at[myrows])
        pltpu.sync_copy(ones16, onesv)
        plsc.subcore_barrier()

        def group(g, carry):
            i0 = g * NB
            da = [pltpu.async_copy(onesv, accd.at[evd.at[i0 + b]],
                                   sems[b], add=True) for b in range(NB)]
            for b in range(NB):
                da[b].wait()
            return carry

        lax.fori_loop(0, nchunk // NB, group, 0)
        plsc.subcore_barrier()
        pltpu.sync_copy(accd.at[myrows], degparts.at[c, myrows])

    return pl.kernel(body, out_type=out_type, mesh=_MESH,
                     scratch_types=tuple(scratch), compiler_params=_UNTILED)


def _dot_t(a, w):
    return lax.dot_general(a, w, (((1,), (1,)), ((), ())),
                           preferred_element_type=jnp.float32)


def _encode_body(x_ref, w_ref, b_ref, o_ref):
    o_ref[...] = jnp.maximum(_dot_t(x_ref[...], w_ref[...]) + b_ref[...], 0.0)


def _combine_body(p_ref, dp_ref, z_ref, wl_ref, bl_ref, wr_ref, o_ref):
    deg = dp_ref[0, :, 0:1] + dp_ref[1, :, 0:1]
    inv = 1.0 / jnp.maximum(deg, 1.0)
    mean = (p_ref[0] + p_ref[1]) * inv
    acc = _dot_t(mean, wl_ref[...]) + bl_ref[...] + _dot_t(z_ref[...], wr_ref[...])
    o_ref[...] = jnp.maximum(acc, 0.0)


def _combine_prep3_body(p_ref, dp_ref, z_ref, wl_ref, bl_ref, wr_ref,
                        wl4_ref, wr4_ref, p3_ref, r3_ref):
    deg = dp_ref[0, :, 0:1] + dp_ref[1, :, 0:1]
    inv = 1.0 / jnp.maximum(deg, 1.0)
    mean = (p_ref[0] + p_ref[1]) * inv
    z3 = jnp.maximum(_dot_t(mean, wl_ref[...]) + bl_ref[...]
                     + _dot_t(z_ref[...], wr_ref[...]), 0.0)
    p3_ref[...] = _dot_t(z3, wl4_ref[...])
    r3_ref[...] = _dot_t(z3, wr4_ref[...])


def _combine3_body(p_ref, dp_ref, r_ref, bl_ref, dw_ref, db_ref, o_ref):
    deg = dp_ref[0, :, 0:1] + dp_ref[1, :, 0:1]
    inv = 1.0 / jnp.maximum(deg, 1.0)
    z4 = (p_ref[0] + p_ref[1]) * inv + bl_ref[...] + r_ref[...]
    o_ref[...] = _dot_t(z4, dw_ref[...]) + db_ref[...]


def _row_spec(width):
    return pl.BlockSpec((ROWS_BLK, width), lambda i: (i, 0))


def _part_spec(width):
    return pl.BlockSpec((NC, ROWS_BLK, width), lambda i: (0, i, 0))


def _full_spec(shape):
    return pl.BlockSpec(shape, lambda i: tuple(0 for _ in shape))


def _tc_call(body, in_specs, out_specs, out_shape):
    return pl.pallas_call(body, grid=(GRID_N,), in_specs=in_specs,
                          out_specs=out_specs, out_shape=out_shape)


@jax.jit
def kernel(x, enc_W, enc_b, l0_Wl, l0_bl, l0_Wr, h1_Wl, h1_bl, h1_Wr,
           h2_Wl, h2_bl, h2_Wr, out_Wl, out_bl, out_Wr, dec_W, dec_b,
           edge_index):
    src_r = edge_index[0].reshape(NW, EPW // B, B)
    dst_r = edge_index[1].reshape(NW, EPW // B, B)
    src_r16 = edge_index[0].reshape(NW, EPW // B16, B16)
    dst_r16 = edge_index[1].reshape(NW, EPW // B16, B16)
    zrows = jnp.zeros((RPS, H), jnp.float32)
    zrows16 = jnp.zeros((RPS, 16), jnp.float32)
    ones16 = jnp.ones((B16, 16), jnp.float32)

    agg128 = _make_sc_agg(H, B)
    agg16 = _make_sc_agg(16, B16)
    deg16 = _make_sc_deg()

    f32 = jnp.float32
    enc = _tc_call(
        _encode_body,
        [_row_spec(H), _full_spec((H, H)), _full_spec((1, H))],
        _row_spec(H), jax.ShapeDtypeStruct((N, H), f32))
    combine = _tc_call(
        _combine_body,
        [_part_spec(H), _part_spec(16), _row_spec(H),
         _full_spec((H, H)), _full_spec((1, H)), _full_spec((H, H))],
        _row_spec(H), jax.ShapeDtypeStruct((N, H), f32))
    combine_prep3 = _tc_call(
        _combine_prep3_body,
        [_part_spec(H), _part_spec(16), _row_spec(H),
         _full_spec((H, H)), _full_spec((1, H)), _full_spec((H, H)),
         _full_spec((16, H)), _full_spec((16, H))],
        [_row_spec(16), _row_spec(16)],
        [jax.ShapeDtypeStruct((N, 16), f32), jax.ShapeDtypeStruct((N, 16), f32)])
    combine3 = _tc_call(
        _combine3_body,
        [_part_spec(16), _part_spec(16), _row_spec(16),
         _full_spec((1, 16)), _full_spec((16, 16)), _full_spec((1, 16))],
        _row_spec(16), jax.ShapeDtypeStruct((N, 16), f32))

    z = enc(x, enc_W, enc_b.reshape(1, H))
    degparts = deg16(dst_r16, zrows16, ones16)
    parts = agg128(z, src_r, dst_r, zrows)
    z = combine(parts, degparts, z, l0_Wl, l0_bl.reshape(1, H), l0_Wr)
    parts = agg128(z, src_r, dst_r, zrows)
    z = combine(parts, degparts, z, h1_Wl, h1_bl.reshape(1, H), h1_Wr)
    parts = agg128(z, src_r, dst_r, zrows)
    wl4 = jnp.zeros((16, H), f32).at[:4].set(out_Wl)
    wr4 = jnp.zeros((16, H), f32).at[:4].set(out_Wr)
    p3, r3 = combine_prep3(parts, degparts, z, h2_Wl, h2_bl.reshape(1, H),
                           h2_Wr, wl4, wr4)
    parts = agg16(p3, src_r16, dst_r16, zrows16)
    bl4 = jnp.zeros((1, 16), f32).at[0, :4].set(out_bl)
    dw = jnp.zeros((16, 16), f32).at[:4, :4].set(dec_W)
    db = jnp.zeros((1, 16), f32).at[0, :4].set(dec_b)
    out16 = combine3(parts, degparts, r3, bl4, dw, db)
    return out16[:, :4]


# speedup vs baseline: 13.8414x; 1.0118x over previous
import jax
import jax.numpy as jnp
from jax import lax
from jax.experimental import pallas as pl
from jax.experimental.pallas import tpu as pltpu
from jax.experimental.pallas import tpu_sc as plsc

N = 10000
E = 320000
H = 128

NC = 2
NS = 16
NW = NC * NS
EPW = E // NW
B = 40
B16 = 80
NP = 10240
RPS = NP // NS

ROWS_BLK = 1000
GRID_N = N // ROWS_BLK


_MESH = plsc.VectorSubcoreMesh(core_axis_name="c", subcore_axis_name="s",
                               num_cores=NC, num_subcores=NS)
_UNTILED = pltpu.CompilerParams(use_tc_tiling_on_sc=False)


NB = 5


def _make_sc_agg(width, bb, nb=NB):
    nchunk = EPW // bb
    out_type = jax.ShapeDtypeStruct((NC, NP, width), jnp.float32)
    scratch = (
        [pltpu.VMEM((nchunk, bb), jnp.int32),
         pltpu.VMEM((nchunk, bb), jnp.int32),
         pltpu.VMEM((nb, bb, width), jnp.float32),
         pltpu.VMEM_SHARED((NP, width), jnp.float32)]
        + [pltpu.SemaphoreType.DMA] * (2 * nb)
    )

    def body(table, src_r, dst_r, zrows, parts, evs, evd, rows, acc, *sems):
        semg, sema = sems[:nb], sems[nb:]
        c = lax.axis_index("c")
        s = lax.axis_index("s")
        w = s * NC + c
        myrows = pl.ds(s * RPS, RPS)
        d1 = pltpu.async_copy(src_r.at[w], evs, semg[0])
        d2 = pltpu.async_copy(dst_r.at[w], evd, semg[1 % nb])
        d3 = pltpu.async_copy(zrows, acc.at[myrows], sema[0])
        d1.wait()
        d2.wait()
        d3.wait()
        plsc.subcore_barrier()

        def group(g, carry):
            i0 = g * nb

            dg = []
            for b in range(nb):
                @pl.when(g > 0)
                def _(b=b):
                    pltpu.make_async_copy(
                        rows.at[b], acc.at[evd.at[i0 - nb + b]],
                        sema[b]).wait()
                dg.append(pltpu.async_copy(table.at[evs.at[i0 + b]],
                                           rows.at[b], semg[b]))
            for b in range(nb):
                dg[b].wait()
                pltpu.async_copy(rows.at[b], acc.at[evd.at[i0 + b]],
                                 sema[b], add=True)
            return carry

        lax.fori_loop(0, nchunk // nb, group, 0)
        for b in range(nb):
            pltpu.make_async_copy(rows.at[b], acc.at[evd.at[nchunk - nb + b]],
                                  sema[b]).wait()
        plsc.subcore_barrier()
        pltpu.sync_copy(acc.at[myrows], parts.at[c, myrows])

    return pl.kernel(body, out_type=out_type, mesh=_MESH,
                     scratch_types=tuple(scratch), compiler_params=_UNTILED)


def _make_sc_deg():
    nchunk = EPW // B16
    out_type = jax.ShapeDtypeStruct((NC, NP, 16), jnp.float32)
    scratch = (
        [pltpu.VMEM((nchunk, B16), jnp.int32),
         pltpu.VMEM((B16, 16), jnp.float32),
         pltpu.VMEM_SHARED((NP, 16), jnp.float32)]
        + [pltpu.SemaphoreType.DMA] * NB
    )

    def body(dst_r, zrows16, ones16, degparts, evd, onesv, accd, *sems):
        c = lax.axis_index("c")
        s = lax.axis_index("s")
        w = s * NC + c
        myrows = pl.ds(s * RPS, RPS)
        d1 = pltpu.async_copy(dst_r.at[w], evd, sems[0])
        d2 = pltpu.async_copy(zrows16, accd.at[myrows], sems[1])
        d3 = pltpu.async_copy(ones16, onesv, sems[2])
        d1.wait()
        d2.wait()
        d3.wait()
        plsc.subcore_barrier()

        def group(g, carry):
            i0 = g * NB
            da = [pltpu.async_copy(onesv, accd.at[evd.at[i0 + b]],
                                   sems[b], add=True) for b in range(NB)]
            for b in range(NB):
                da[b].wait()
            return carry

        lax.fori_loop(0, nchunk // NB, group, 0)
        plsc.subcore_barrier()
        pltpu.sync_copy(accd.at[myrows], degparts.at[c, myrows])

    return pl.kernel(body, out_type=out_type, mesh=_MESH,
                     scratch_types=tuple(scratch), compiler_params=_UNTILED)


def _dot_t(a, w):
    return lax.dot_general(a, w, (((1,), (1,)), ((), ())),
                           preferred_element_type=jnp.float32)


def _encode_body(x_ref, w_ref, b_ref, o_ref):
    o_ref[...] = jnp.maximum(_dot_t(x_ref[...], w_ref[...]) + b_ref[...], 0.0)


def _combine_body(p_ref, dp_ref, z_ref, wl_ref, bl_ref, wr_ref, o_ref):
    deg = dp_ref[0, :, 0:1] + dp_ref[1, :, 0:1]
    inv = 1.0 / jnp.maximum(deg, 1.0)
    mean = (p_ref[0] + p_ref[1]) * inv
    acc = _dot_t(mean, wl_ref[...]) + bl_ref[...] + _dot_t(z_ref[...], wr_ref[...])
    o_ref[...] = jnp.maximum(acc, 0.0)


def _combine_prep3_body(p_ref, dp_ref, z_ref, wl_ref, bl_ref, wr_ref,
                        wl4_ref, wr4_ref, p3_ref, r3_ref):
    deg = dp_ref[0, :, 0:1] + dp_ref[1, :, 0:1]
    inv = 1.0 / jnp.maximum(deg, 1.0)
    mean = (p_ref[0] + p_ref[1]) * inv
    z3 = jnp.maximum(_dot_t(mean, wl_ref[...]) + bl_ref[...]
                     + _dot_t(z_ref[...], wr_ref[...]), 0.0)
    p3_ref[...] = _dot_t(z3, wl4_ref[...])
    r3_ref[...] = _dot_t(z3, wr4_ref[...])


def _combine3_body(p_ref, dp_ref, r_ref, bl_ref, dw_ref, db_ref, o_ref):
    deg = dp_ref[0, :, 0:1] + dp_ref[1, :, 0:1]
    inv = 1.0 / jnp.maximum(deg, 1.0)
    z4 = (p_ref[0] + p_ref[1]) * inv + bl_ref[...] + r_ref[...]
    o_ref[...] = _dot_t(z4, dw_ref[...]) + db_ref[...]


def _row_spec(width):
    return pl.BlockSpec((ROWS_BLK, width), lambda i: (i, 0))


def _part_spec(width):
    return pl.BlockSpec((NC, ROWS_BLK, width), lambda i: (0, i, 0))


def _full_spec(shape):
    return pl.BlockSpec(shape, lambda i: tuple(0 for _ in shape))


def _tc_call(body, in_specs, out_specs, out_shape):
    return pl.pallas_call(body, grid=(GRID_N,), in_specs=in_specs,
                          out_specs=out_specs, out_shape=out_shape)


@jax.jit
def kernel(x, enc_W, enc_b, l0_Wl, l0_bl, l0_Wr, h1_Wl, h1_bl, h1_Wr,
           h2_Wl, h2_bl, h2_Wr, out_Wl, out_bl, out_Wr, dec_W, dec_b,
           edge_index):
    src_r = edge_index[0].reshape(NW, EPW // B, B)
    dst_r = edge_index[1].reshape(NW, EPW // B, B)
    src_r16 = edge_index[0].reshape(NW, EPW // B16, B16)
    dst_r16 = edge_index[1].reshape(NW, EPW // B16, B16)
    zrows = jnp.zeros((RPS, H), jnp.float32)
    zrows16 = jnp.zeros((RPS, 16), jnp.float32)
    ones16 = jnp.ones((B16, 16), jnp.float32)

    agg128 = _make_sc_agg(H, B)
    agg16 = _make_sc_agg(16, B16)
    deg16 = _make_sc_deg()

    f32 = jnp.float32
    enc = _tc_call(
        _encode_body,
        [_row_spec(H), _full_spec((H, H)), _full_spec((1, H))],
        _row_spec(H), jax.ShapeDtypeStruct((N, H), f32))
    combine = _tc_call(
        _combine_body,
        [_part_spec(H), _part_spec(16), _row_spec(H),
         _full_spec((H, H)), _full_spec((1, H)), _full_spec((H, H))],
        _row_spec(H), jax.ShapeDtypeStruct((N, H), f32))
    combine_prep3 = _tc_call(
        _combine_prep3_body,
        [_part_spec(H), _part_spec(16), _row_spec(H),
         _full_spec((H, H)), _full_spec((1, H)), _full_spec((H, H)),
         _full_spec((16, H)), _full_spec((16, H))],
        [_row_spec(16), _row_spec(16)],
        [jax.ShapeDtypeStruct((N, 16), f32), jax.ShapeDtypeStruct((N, 16), f32)])
    combine3 = _tc_call(
        _combine3_body,
        [_part_spec(16), _part_spec(16), _row_spec(16),
         _full_spec((1, 16)), _full_spec((16, 16)), _full_spec((1, 16))],
        _row_spec(16), jax.ShapeDtypeStruct((N, 16), f32))

    z = enc(x, enc_W, enc_b.reshape(1, H))
    degparts = deg16(dst_r16, zrows16, ones16)
    parts = agg128(z, src_r, dst_r, zrows)
    z = combine(parts, degparts, z, l0_Wl, l0_bl.reshape(1, H), l0_Wr)
    parts = agg128(z, src_r, dst_r, zrows)
    z = combine(parts, degparts, z, h1_Wl, h1_bl.reshape(1, H), h1_Wr)
    parts = agg128(z, src_r, dst_r, zrows)
    wl4 = jnp.zeros((16, H), f32).at[:4].set(out_Wl)
    wr4 = jnp.zeros((16, H), f32).at[:4].set(out_Wr)
    p3, r3 = combine_prep3(parts, degparts, z, h2_Wl, h2_bl.reshape(1, H),
                           h2_Wr, wl4, wr4)
    parts = agg16(p3, src_r16, dst_r16, zrows16)
    bl4 = jnp.zeros((1, 16), f32).at[0, :4].set(out_bl)
    dw = jnp.zeros((16, 16), f32).at[:4, :4].set(dec_W)
    db = jnp.zeros((1, 16), f32).at[0, :4].set(dec_b)
    out16 = combine3(parts, degparts, r3, bl4, dw, db)
    return out16[:, :4]


# speedup vs baseline: 13.8625x; 1.0015x over previous
import jax
import jax.numpy as jnp
from jax import lax
from jax.experimental import pallas as pl
from jax.experimental.pallas import tpu as pltpu
from jax.experimental.pallas import tpu_sc as plsc

N = 10000
E = 320000
H = 128

NC = 2
NS = 16
NW = NC * NS
EPW = E // NW
B = 40
B16 = 80
NP = 10240
RPS = NP // NS

ROWS_BLK = 1000
GRID_N = N // ROWS_BLK


_MESH = plsc.VectorSubcoreMesh(core_axis_name="c", subcore_axis_name="s",
                               num_cores=NC, num_subcores=NS)
_UNTILED = pltpu.CompilerParams(use_tc_tiling_on_sc=False)


NB = 5


def _make_sc_agg(width, bb, nb=NB):
    nchunk = EPW // bb
    out_type = jax.ShapeDtypeStruct((NC, NP, width), jnp.float32)
    scratch = (
        [pltpu.VMEM((nchunk, bb), jnp.int32),
         pltpu.VMEM((nchunk, bb), jnp.int32),
         pltpu.VMEM((nb, bb, width), jnp.float32),
         pltpu.VMEM_SHARED((NP, width), jnp.float32)]
        + [pltpu.SemaphoreType.DMA] * (2 * nb)
    )

    def body(table, src_r, dst_r, zrows, parts, evs, evd, rows, acc, *sems):
        semg, sema = sems[:nb], sems[nb:]
        c = lax.axis_index("c")
        s = lax.axis_index("s")
        w = s * NC + c
        myrows = pl.ds(s * RPS, RPS)
        d1 = pltpu.async_copy(src_r.at[w], evs, semg[0])
        d2 = pltpu.async_copy(dst_r.at[w], evd, semg[1 % nb])
        d3 = pltpu.async_copy(zrows, acc.at[myrows], sema[0])
        d1.wait()
        d2.wait()
        d3.wait()
        plsc.subcore_barrier()

        def group(g, carry):
            i0 = g * nb

            dg = []
            for b in range(nb):
                @pl.when(g > 0)
                def _(b=b):
                    pltpu.make_async_copy(
                        rows.at[b], acc.at[evd.at[i0 - nb + b]],
                        sema[b]).wait()
                dg.append(pltpu.async_copy(table.at[evs.at[i0 + b]],
                                           rows.at[b], semg[b]))
            for b in range(nb):
                dg[b].wait()
                pltpu.async_copy(rows.at[b], acc.at[evd.at[i0 + b]],
                                 sema[b], add=True)
            return carry

        lax.fori_loop(0, nchunk // nb, group, 0)
        for b in range(nb):
            pltpu.make_async_copy(rows.at[b], acc.at[evd.at[nchunk - nb + b]],
                                  sema[b]).wait()
        plsc.subcore_barrier()
        pltpu.sync_copy(acc.at[myrows], parts.at[c, myrows])

    return pl.kernel(body, out_type=out_type, mesh=_MESH,
                     scratch_types=tuple(scratch), compiler_params=_UNTILED)


def _make_sc_deg():
    nchunk = EPW // B16
    out_type = jax.ShapeDtypeStruct((NC, NP, 16), jnp.float32)
    scratch = (
        [pltpu.VMEM((nchunk, B16), jnp.int32),
         pltpu.VMEM((B16, 16), jnp.float32),
         pltpu.VMEM_SHARED((NP, 16), jnp.float32)]
        + [pltpu.SemaphoreType.DMA] * NB
    )

    def body(dst_r, zrows16, ones16, degparts, evd, onesv, accd, *sems):
        c = lax.axis_index("c")
        s = lax.axis_index("s")
        w = s * NC + c
        myrows = pl.ds(s * RPS, RPS)
        d1 = pltpu.async_copy(dst_r.at[w], evd, sems[0])
        d2 = pltpu.async_copy(zrows16, accd.at[myrows], sems[1])
        d3 = pltpu.async_copy(ones16, onesv, sems[2])
        d1.wait()
        d2.wait()
        d3.wait()
        plsc.subcore_barrier()

        def group(g, carry):
            i0 = g * NB
            for b in range(NB):
                @pl.when(g > 0)
                def _(b=b):
                    pltpu.make_async_copy(onesv, accd.at[evd.at[i0 - NB + b]],
                                          sems[b]).wait()
                pltpu.async_copy(onesv, accd.at[evd.at[i0 + b]],
                                 sems[b], add=True)
            return carry

        lax.fori_loop(0, nchunk // NB, group, 0)
        for b in range(NB):
            pltpu.make_async_copy(onesv, accd.at[evd.at[nchunk - NB + b]],
                                  sems[b]).wait()
        plsc.subcore_barrier()
        pltpu.sync_copy(accd.at[myrows], degparts.at[c, myrows])

    return pl.kernel(body, out_type=out_type, mesh=_MESH,
                     scratch_types=tuple(scratch), compiler_params=_UNTILED)


def _dot_t(a, w):
    return lax.dot_general(a, w, (((1,), (1,)), ((), ())),
                           preferred_element_type=jnp.float32)


def _encode_body(x_ref, w_ref, b_ref, o_ref):
    o_ref[...] = jnp.maximum(_dot_t(x_ref[...], w_ref[...]) + b_ref[...], 0.0)


def _combine_body(p_ref, dp_ref, z_ref, wl_ref, bl_ref, wr_ref, o_ref):
    deg = dp_ref[0, :, 0:1] + dp_ref[1, :, 0:1]
    inv = 1.0 / jnp.maximum(deg, 1.0)
    mean = (p_ref[0] + p_ref[1]) * inv
    acc = _dot_t(mean, wl_ref[...]) + bl_ref[...] + _dot_t(z_ref[...], wr_ref[...])
    o_ref[...] = jnp.maximum(acc, 0.0)


def _combine_prep3_body(p_ref, dp_ref, z_ref, wl_ref, bl_ref, wr_ref,
                        wl4_ref, wr4_ref, p3_ref, r3_ref):
    deg = dp_ref[0, :, 0:1] + dp_ref[1, :, 0:1]
    inv = 1.0 / jnp.maximum(deg, 1.0)
    mean = (p_ref[0] + p_ref[1]) * inv
    z3 = jnp.maximum(_dot_t(mean, wl_ref[...]) + bl_ref[...]
                     + _dot_t(z_ref[...], wr_ref[...]), 0.0)
    p3_ref[...] = _dot_t(z3, wl4_ref[...])
    r3_ref[...] = _dot_t(z3, wr4_ref[...])


def _combine3_body(p_ref, dp_ref, r_ref, bl_ref, dw_ref, db_ref, o_ref):
    deg = dp_ref[0, :, 0:1] + dp_ref[1, :, 0:1]
    inv = 1.0 / jnp.maximum(deg, 1.0)
    z4 = (p_ref[0] + p_ref[1]) * inv + bl_ref[...] + r_ref[...]
    o_ref[...] = _dot_t(z4, dw_ref[...]) + db_ref[...]


def _row_spec(width):
    return pl.BlockSpec((ROWS_BLK, width), lambda i: (i, 0))


def _part_spec(width):
    return pl.BlockSpec((NC, ROWS_BLK, width), lambda i: (0, i, 0))


def _full_spec(shape):
    return pl.BlockSpec(shape, lambda i: tuple(0 for _ in shape))


def _tc_call(body, in_specs, out_specs, out_shape):
    return pl.pallas_call(body, grid=(GRID_N,), in_specs=in_specs,
                          out_specs=out_specs, out_shape=out_shape)


@jax.jit
def kernel(x, enc_W, enc_b, l0_Wl, l0_bl, l0_Wr, h1_Wl, h1_bl, h1_Wr,
           h2_Wl, h2_bl, h2_Wr, out_Wl, out_bl, out_Wr, dec_W, dec_b,
           edge_index):
    src_r = edge_index[0].reshape(NW, EPW // B, B)
    dst_r = edge_index[1].reshape(NW, EPW // B, B)
    src_r16 = edge_index[0].reshape(NW, EPW // B16, B16)
    dst_r16 = edge_index[1].reshape(NW, EPW // B16, B16)
    zrows = jnp.zeros((RPS, H), jnp.float32)
    zrows16 = jnp.zeros((RPS, 16), jnp.float32)
    ones16 = jnp.ones((B16, 16), jnp.float32)

    agg128 = _make_sc_agg(H, B)
    agg16 = _make_sc_agg(16, B16)
    deg16 = _make_sc_deg()

    f32 = jnp.float32
    enc = _tc_call(
        _encode_body,
        [_row_spec(H), _full_spec((H, H)), _full_spec((1, H))],
        _row_spec(H), jax.ShapeDtypeStruct((N, H), f32))
    combine = _tc_call(
        _combine_body,
        [_part_spec(H), _part_spec(16), _row_spec(H),
         _full_spec((H, H)), _full_spec((1, H)), _full_spec((H, H))],
        _row_spec(H), jax.ShapeDtypeStruct((N, H), f32))
    combine_prep3 = _tc_call(
        _combine_prep3_body,
        [_part_spec(H), _part_spec(16), _row_spec(H),
         _full_spec((H, H)), _full_spec((1, H)), _full_spec((H, H)),
         _full_spec((16, H)), _full_spec((16, H))],
        [_row_spec(16), _row_spec(16)],
        [jax.ShapeDtypeStruct((N, 16), f32), jax.ShapeDtypeStruct((N, 16), f32)])
    combine3 = _tc_call(
        _combine3_body,
        [_part_spec(16), _part_spec(16), _row_spec(16),
         _full_spec((1, 16)), _full_spec((16, 16)), _full_spec((1, 16))],
        _row_spec(16), jax.ShapeDtypeStruct((N, 16), f32))

    z = enc(x, enc_W, enc_b.reshape(1, H))
    degparts = deg16(dst_r16, zrows16, ones16)
    parts = agg128(z, src_r, dst_r, zrows)
    z = combine(parts, degparts, z, l0_Wl, l0_bl.reshape(1, H), l0_Wr)
    parts = agg128(z, src_r, dst_r, zrows)
    z = combine(parts, degparts, z, h1_Wl, h1_bl.reshape(1, H), h1_Wr)
    parts = agg128(z, src_r, dst_r, zrows)
    wl4 = jnp.zeros((16, H), f32).at[:4].set(out_Wl)
    wr4 = jnp.zeros((16, H), f32).at[:4].set(out_Wr)
    p3, r3 = combine_prep3(parts, degparts, z, h2_Wl, h2_bl.reshape(1, H),
                           h2_Wr, wl4, wr4)
    parts = agg16(p3, src_r16, dst_r16, zrows16)
    bl4 = jnp.zeros((1, 16), f32).at[0, :4].set(out_bl)
    dw = jnp.zeros((16, 16), f32).at[:4, :4].set(dec_W)
    db = jnp.zeros((1, 16), f32).at[0, :4].set(dec_b)
    out16 = combine3(parts, degparts, r3, bl4, dw, db)
    return out16[:, :4]
